# Initial kernel scaffold; baseline (speedup 1.0000x reference)
import jax
import jax.numpy as jnp
from jax.experimental import pallas as pl


def kernel(x, edge_index, edge_attr, W1, b1, g1, bt1, W2, b2, g2, bt2, fcW, fcb, fc1W, fc1b):
    raise NotImplementedError("write your pallas kernel here")


# speedup vs baseline: 1.2948x; 1.2948x over previous
import jax
import jax.numpy as jnp
from jax.experimental import pallas as pl
from jax.experimental.pallas import tpu as pltpu


def _pick_block(n):
    for blk in (1000, 400, 200, 100, 8):
        if n % blk == 0 and blk % 8 == 0:
            return blk
    return n


def _mlp_kernel(n_rows, blk, eps,
                x_ref, W1_ref, b1_ref, g1_ref, bt1_ref,
                W2_ref, b2_ref, g2_ref, bt2_ref,
                fcW_ref, fcb_ref, fc1W_ref, fc1b_ref,
                out_ref, h2_scr, s1, q1, s2, q2):
    p = pl.program_id(0)
    i = pl.program_id(1)
    inv_n = 1.0 / n_rows

    @pl.when(p == 0)
    def _phase0():
        h1 = jnp.dot(x_ref[...], W1_ref[...],
                     preferred_element_type=jnp.float32) + b1_ref[...]

        @pl.when(i == 0)
        def _init():
            s1[...] = jnp.zeros_like(s1)
            q1[...] = jnp.zeros_like(q1)

        s1[...] += h1.sum(axis=0, keepdims=True)
        q1[...] += (h1 * h1).sum(axis=0, keepdims=True)

    @pl.when(p == 1)
    def _phase1():
        h1 = jnp.dot(x_ref[...], W1_ref[...],
                     preferred_element_type=jnp.float32) + b1_ref[...]
        mean1 = s1[...] * inv_n
        var1 = q1[...] * inv_n - mean1 * mean1
        inv1 = jax.lax.rsqrt(var1 + eps)
        u = jnp.maximum((h1 - mean1) * inv1 * g1_ref[...] + bt1_ref[...], 0.0)
        h2 = jnp.dot(u, W2_ref[...],
                     preferred_element_type=jnp.float32) + b2_ref[...]
        h2_scr[pl.ds(i * blk, blk), :] = h2

        @pl.when(i == 0)
        def _init():
            s2[...] = jnp.zeros_like(s2)
            q2[...] = jnp.zeros_like(q2)

        s2[...] += h2.sum(axis=0, keepdims=True)
        q2[...] += (h2 * h2).sum(axis=0, keepdims=True)

    @pl.when(p == 2)
    def _phase2():
        h2 = h2_scr[pl.ds(i * blk, blk), :]
        mean2 = s2[...] * inv_n
        var2 = q2[...] * inv_n - mean2 * mean2
        inv2 = jax.lax.rsqrt(var2 + eps)
        v = jnp.maximum((h2 - mean2) * inv2 * g2_ref[...] + bt2_ref[...], 0.0)
        a = jnp.maximum(jnp.dot(v, fcW_ref[...],
                                preferred_element_type=jnp.float32)
                        + fcb_ref[...], 0.0)
        out_ref[...] = jnp.dot(a, fc1W_ref[...],
                               preferred_element_type=jnp.float32) + fc1b_ref[...]


def kernel(x, edge_index, edge_attr, W1, b1, g1, bt1, W2, b2, g2, bt2,
           fcW, fcb, fc1W, fc1b):
    del edge_index, edge_attr
    N, F = x.shape
    H = W1.shape[1]
    S = fcW.shape[1]
    O = fc1W.shape[1]
    OP = 128
    blk = _pick_block(N)
    nb = N // blk

    fc1W_p = jnp.zeros((S, OP), jnp.float32).at[:, :O].set(fc1W)
    fc1b_p = jnp.zeros((1, OP), jnp.float32).at[0, :O].set(fc1b)

    row2 = lambda v: v.reshape(1, -1)
    full = lambda shape: pl.BlockSpec(shape, lambda p, i: (0, 0))

    out = pl.pallas_call(
        lambda *refs: _mlp_kernel(N, blk, 1e-5, *refs),
        grid=(3, nb),
        in_specs=[
            pl.BlockSpec((blk, F), lambda p, i: (i, 0)),
            full((F, H)), full((1, H)), full((1, H)), full((1, H)),
            full((H, H)), full((1, H)), full((1, H)), full((1, H)),
            full((H, S)), full((1, S)),
            full((S, OP)), full((1, OP)),
        ],
        out_specs=pl.BlockSpec((blk, OP), lambda p, i: (i, 0)),
        out_shape=jax.ShapeDtypeStruct((N, OP), jnp.float32),
        scratch_shapes=[
            pltpu.VMEM((N, H), jnp.float32),
            pltpu.VMEM((1, H), jnp.float32),
            pltpu.VMEM((1, H), jnp.float32),
            pltpu.VMEM((1, H), jnp.float32),
            pltpu.VMEM((1, H), jnp.float32),
        ],
        compiler_params=pltpu.CompilerParams(
            dimension_semantics=("arbitrary", "arbitrary"),
            vmem_limit_bytes=100 * 1024 * 1024),
    )(x, W1, row2(b1), row2(g1), row2(bt1),
      W2, row2(b2), row2(g2), row2(bt2),
      fcW, row2(fcb), fc1W_p, fc1b_p)
    return out[:, :O]


# speedup vs baseline: 1.3802x; 1.0659x over previous
import jax
import jax.numpy as jnp
from jax.experimental import pallas as pl
from jax.experimental.pallas import tpu as pltpu


def _pick_block(n):
    for blk in (1000, 400, 200, 100, 8):
        if n % blk == 0 and blk % 8 == 0:
            return blk
    return n


def _mlp_kernel(n_rows, blk, eps,
                x_ref, W1_ref, b1_ref, g1_ref, bt1_ref,
                W2_ref, b2_ref, g2_ref, bt2_ref,
                fcW_ref, fcb_ref, fc1W_ref, fc1b_ref,
                out_ref, h2_scr, gram, xs, s1, q1, s2, q2):
    p = pl.program_id(0)
    i = pl.program_id(1)
    inv_n = 1.0 / n_rows

    @pl.when(p == 0)
    def _phase0():
        xb = x_ref[...]

        @pl.when(i == 0)
        def _init():
            gram[...] = jnp.zeros_like(gram)
            xs[...] = jnp.zeros_like(xs)

        gram[...] += jax.lax.dot_general(
            xb, xb, (((0,), (0,)), ((), ())),
            preferred_element_type=jnp.float32)
        xs[...] += xb.sum(axis=0, keepdims=True)

    @pl.when(p == 1)
    def _phase1():
        @pl.when(i == 0)
        def _finalize_stats():
            m0 = jnp.dot(xs[...], W1_ref[...],
                         preferred_element_type=jnp.float32) * inv_n
            gw = jnp.dot(gram[...], W1_ref[...],
                         preferred_element_type=jnp.float32)
            q0 = (W1_ref[...] * gw).sum(axis=0, keepdims=True) * inv_n
            s1[...] = m0 + b1_ref[...]
            q1[...] = q0 - m0 * m0

        h1 = jnp.dot(x_ref[...], W1_ref[...],
                     preferred_element_type=jnp.float32) + b1_ref[...]
        inv1 = jax.lax.rsqrt(q1[...] + eps)
        u = jnp.maximum((h1 - s1[...]) * inv1 * g1_ref[...] + bt1_ref[...], 0.0)
        h2 = jnp.dot(u, W2_ref[...],
                     preferred_element_type=jnp.float32) + b2_ref[...]
        h2_scr[pl.ds(i * blk, blk), :] = h2

        @pl.when(i == 0)
        def _init():
            s2[...] = jnp.zeros_like(s2)
            q2[...] = jnp.zeros_like(q2)

        s2[...] += h2.sum(axis=0, keepdims=True)
        q2[...] += (h2 * h2).sum(axis=0, keepdims=True)

    @pl.when(p == 2)
    def _phase2():
        h2 = h2_scr[pl.ds(i * blk, blk), :]
        mean2 = s2[...] * inv_n
        var2 = q2[...] * inv_n - mean2 * mean2
        inv2 = jax.lax.rsqrt(var2 + eps)
        v = jnp.maximum((h2 - mean2) * inv2 * g2_ref[...] + bt2_ref[...], 0.0)
        a = jnp.maximum(jnp.dot(v, fcW_ref[...],
                                preferred_element_type=jnp.float32)
                        + fcb_ref[...], 0.0)
        out_ref[...] = jnp.dot(a, fc1W_ref[...],
                               preferred_element_type=jnp.float32) + fc1b_ref[...]


def kernel(x, edge_index, edge_attr, W1, b1, g1, bt1, W2, b2, g2, bt2,
           fcW, fcb, fc1W, fc1b):
    del edge_index, edge_attr
    N, F = x.shape
    H = W1.shape[1]
    S = fcW.shape[1]
    O = fc1W.shape[1]
    OP = 128
    blk = _pick_block(N)
    nb = N // blk

    fc1W_p = jnp.zeros((S, OP), jnp.float32).at[:, :O].set(fc1W)
    fc1b_p = jnp.zeros((1, OP), jnp.float32).at[0, :O].set(fc1b)

    row2 = lambda v: v.reshape(1, -1)
    full = lambda shape: pl.BlockSpec(shape, lambda p, i: (0, 0))

    out = pl.pallas_call(
        lambda *refs: _mlp_kernel(N, blk, 1e-5, *refs),
        grid=(3, nb),
        in_specs=[
            pl.BlockSpec((blk, F), lambda p, i: (i, 0)),
            full((F, H)), full((1, H)), full((1, H)), full((1, H)),
            full((H, H)), full((1, H)), full((1, H)), full((1, H)),
            full((H, S)), full((1, S)),
            full((S, OP)), full((1, OP)),
        ],
        out_specs=pl.BlockSpec((blk, OP), lambda p, i: (i, 0)),
        out_shape=jax.ShapeDtypeStruct((N, OP), jnp.float32),
        scratch_shapes=[
            pltpu.VMEM((N, H), jnp.float32),
            pltpu.VMEM((F, F), jnp.float32),
            pltpu.VMEM((1, F), jnp.float32),
            pltpu.VMEM((1, H), jnp.float32),
            pltpu.VMEM((1, H), jnp.float32),
            pltpu.VMEM((1, H), jnp.float32),
            pltpu.VMEM((1, H), jnp.float32),
        ],
        compiler_params=pltpu.CompilerParams(
            dimension_semantics=("arbitrary", "arbitrary"),
            vmem_limit_bytes=100 * 1024 * 1024),
    )(x, W1, row2(b1), row2(g1), row2(bt1),
      W2, row2(b2), row2(g2), row2(bt2),
      fcW, row2(fcb), fc1W_p, fc1b_p)
    return out[:, :O]


# speedup vs baseline: 1.3844x; 1.0031x over previous
import jax
import jax.numpy as jnp
from jax.experimental import pallas as pl
from jax.experimental.pallas import tpu as pltpu


def _pick_block(n):
    for blk in (1000, 400, 200, 100, 8):
        if n % blk == 0 and blk % 8 == 0:
            return blk
    return n


def _mlp_kernel(n_rows, blk, eps,
                x_ref, W1_ref, b1_ref, g1_ref, bt1_ref,
                W2_ref, b2_ref, g2_ref, bt2_ref,
                fcW_ref, fcb_ref, fc1W_ref, fc1b_ref,
                out_ref, h2_scr, gram, xs, s1, q1, s2, q2, W1s):
    p = pl.program_id(0)
    i = pl.program_id(1)
    inv_n = 1.0 / n_rows

    @pl.when(p == 0)
    def _phase0():
        xb = x_ref[...]

        @pl.when(i == 0)
        def _init():
            gram[...] = jnp.zeros_like(gram)
            xs[...] = jnp.zeros_like(xs)

        gram[...] += jax.lax.dot_general(
            xb, xb, (((0,), (0,)), ((), ())),
            preferred_element_type=jnp.float32)
        xs[...] += xb.sum(axis=0, keepdims=True)

    @pl.when(p == 1)
    def _phase1():
        @pl.when(i == 0)
        def _finalize_stats():
            m0 = jnp.dot(xs[...], W1_ref[...],
                         preferred_element_type=jnp.float32) * inv_n
            gw = jnp.dot(gram[...], W1_ref[...],
                         preferred_element_type=jnp.float32)
            q0 = (W1_ref[...] * gw).sum(axis=0, keepdims=True) * inv_n
            var1 = q0 - m0 * m0
            s = jax.lax.rsqrt(var1 + eps) * g1_ref[...]
            W1s[...] = W1_ref[...] * s
            q1[...] = bt1_ref[...] - m0 * s

        u = jnp.maximum(jnp.dot(x_ref[...], W1s[...],
                                preferred_element_type=jnp.float32)
                        + q1[...], 0.0)
        h2 = jnp.dot(u, W2_ref[...],
                     preferred_element_type=jnp.float32) + b2_ref[...]
        h2_scr[pl.ds(i * blk, blk), :] = h2

        @pl.when(i == 0)
        def _init():
            s2[...] = jnp.zeros_like(s2)
            q2[...] = jnp.zeros_like(q2)

        s2[...] += h2.sum(axis=0, keepdims=True)
        q2[...] += (h2 * h2).sum(axis=0, keepdims=True)

    @pl.when(p == 2)
    def _phase2():
        @pl.when(i == 0)
        def _finalize_stats():
            mean2 = s2[...] * inv_n
            var2 = q2[...] * inv_n - mean2 * mean2
            sv = jax.lax.rsqrt(var2 + eps) * g2_ref[...]
            s2[...] = sv
            q2[...] = bt2_ref[...] - mean2 * sv

        h2 = h2_scr[pl.ds(i * blk, blk), :]
        v = jnp.maximum(h2 * s2[...] + q2[...], 0.0)
        a = jnp.maximum(jnp.dot(v, fcW_ref[...],
                                preferred_element_type=jnp.float32)
                        + fcb_ref[...], 0.0)
        out_ref[...] = jnp.dot(a, fc1W_ref[...],
                               preferred_element_type=jnp.float32) + fc1b_ref[...]


def kernel(x, edge_index, edge_attr, W1, b1, g1, bt1, W2, b2, g2, bt2,
           fcW, fcb, fc1W, fc1b):
    del edge_index, edge_attr
    N, F = x.shape
    H = W1.shape[1]
    S = fcW.shape[1]
    O = fc1W.shape[1]
    OP = 128
    blk = _pick_block(N)
    nb = N // blk

    fc1W_p = jnp.zeros((S, OP), jnp.float32).at[:, :O].set(fc1W)
    fc1b_p = jnp.zeros((1, OP), jnp.float32).at[0, :O].set(fc1b)

    row2 = lambda v: v.reshape(1, -1)
    full = lambda shape: pl.BlockSpec(shape, lambda p, i: (0, 0))

    out = pl.pallas_call(
        lambda *refs: _mlp_kernel(N, blk, 1e-5, *refs),
        grid=(3, nb),
        in_specs=[
            pl.BlockSpec((blk, F), lambda p, i: (i, 0)),
            full((F, H)), full((1, H)), full((1, H)), full((1, H)),
            full((H, H)), full((1, H)), full((1, H)), full((1, H)),
            full((H, S)), full((1, S)),
            full((S, OP)), full((1, OP)),
        ],
        out_specs=pl.BlockSpec((blk, OP), lambda p, i: (i, 0)),
        out_shape=jax.ShapeDtypeStruct((N, OP), jnp.float32),
        scratch_shapes=[
            pltpu.VMEM((N, H), jnp.float32),
            pltpu.VMEM((F, F), jnp.float32),
            pltpu.VMEM((1, F), jnp.float32),
            pltpu.VMEM((1, H), jnp.float32),
            pltpu.VMEM((1, H), jnp.float32),
            pltpu.VMEM((1, H), jnp.float32),
            pltpu.VMEM((1, H), jnp.float32),
            pltpu.VMEM((F, H), jnp.float32),
        ],
        compiler_params=pltpu.CompilerParams(
            dimension_semantics=("arbitrary", "arbitrary"),
            vmem_limit_bytes=100 * 1024 * 1024),
    )(x, W1, row2(b1), row2(g1), row2(bt1),
      W2, row2(b2), row2(g2), row2(bt2),
      fcW, row2(fcb), fc1W_p, fc1b_p)
    return out[:, :O]
